# Initial kernel scaffold; baseline (speedup 1.0000x reference)
import jax
import jax.numpy as jnp
from jax.experimental import pallas as pl


def kernel(image, object_emb, color_emb, state_emb):
    raise NotImplementedError("write your pallas kernel here")


# speedup vs baseline: 2.2604x; 2.2604x over previous
import functools

import jax
import jax.numpy as jnp
from jax import lax
from jax.experimental import pallas as pl
from jax.experimental.pallas import tpu as pltpu, tpu_sc as plsc

BATCH, H, W = 1024, 32, 32
EMBED_DIM = 64
VOCAB = 1000
HW = H * W
NPOS = BATCH * HW

NUM_CORES = 2
NUM_SUBCORES = 16
LANES = 16

D_HALF = EMBED_DIM // NUM_CORES
POS_PER_SUB = NPOS // NUM_SUBCORES
CHUNK = 256
NCH = POS_PER_SUB // CHUNK
GROUPS = CHUNK // LANES
ID_WORDS = CHUNK * 3


def _body(img_hbm, obj_hbm, col_hbm, st_hbm, out_hbm,
          tob, tcb, tsb, idb0, idb1, ob0, ob1, sem_in, sem_out):
    idbufs = (idb0, idb1)
    obufs = (ob0, ob1)
    c = lax.axis_index("c")
    s = lax.axis_index("s")
    dbase = c * D_HALF
    pos_base = s * POS_PER_SUB

    pltpu.sync_copy(obj_hbm.at[pl.ds(dbase, D_HALF), :], tob)
    pltpu.sync_copy(col_hbm.at[pl.ds(dbase, D_HALF), :], tcb)
    pltpu.sync_copy(st_hbm.at[pl.ds(dbase, D_HALF), :], tsb)

    def id_dma(i, slot):
        off = (pos_base + i * CHUNK) * 3
        return pltpu.make_async_copy(
            img_hbm.at[pl.ds(off, ID_WORDS)], idbufs[slot], sem_in)

    def out_dma(i, slot):
        pos = pos_base + i * CHUNK
        b = pos // HW
        po = pos % HW
        return pltpu.make_async_copy(
            obufs[slot],
            out_hbm.at[b, pl.ds(dbase, D_HALF), pl.ds(po, CHUNK)],
            sem_out)

    iota = lax.iota(jnp.int32, LANES)
    i3 = iota * 3

    def compute_chunk(slot):
        idb = idbufs[slot]
        ob = obufs[slot]

        def g_body(g, _):
            base = g * (3 * LANES)
            idx_o = i3 + base
            ids_o = plsc.load_gather(idb, [idx_o])
            ids_c = plsc.load_gather(idb, [idx_o + 1])
            ids_s = plsc.load_gather(idb, [idx_o + 2])
            for d in range(D_HALF):
                dsp = jnp.full((LANES,), d, jnp.int32)
                vo = plsc.load_gather(tob, [dsp, ids_o])
                vc = plsc.load_gather(tcb, [dsp, ids_c])
                vs = plsc.load_gather(tsb, [dsp, ids_s])
                ob[d, pl.ds(g * LANES, LANES)] = \
                    (vo + vc + vs) * jnp.float32(1.0 / 3.0)
            return 0
        lax.fori_loop(0, GROUPS, g_body, 0, unroll=False)

    id_dma(0, 0).start()

    def step(k, _):
        for slot in (0, 1):
            i = k * 2 + slot
            id_dma(i, slot).wait()

            @pl.when(i < NCH - 1)
            def _prefetch():
                id_dma(i + 1, 1 - slot).start()

            @pl.when(i >= 2)
            def _drain():
                out_dma(i - 2, slot).wait()

            compute_chunk(slot)
            out_dma(i, slot).start()
        return 0

    lax.fori_loop(0, NCH // 2, step, 0, unroll=False)
    out_dma(NCH - 2, 0).wait()
    out_dma(NCH - 1, 1).wait()


@functools.partial(
    pl.kernel,
    out_type=jax.ShapeDtypeStruct((BATCH, EMBED_DIM, HW), jnp.float32),
    mesh=plsc.VectorSubcoreMesh(core_axis_name="c", subcore_axis_name="s",
                                num_cores=NUM_CORES,
                                num_subcores=NUM_SUBCORES),
    scratch_types=[
        pltpu.VMEM((D_HALF, VOCAB), jnp.float32),
        pltpu.VMEM((D_HALF, VOCAB), jnp.float32),
        pltpu.VMEM((D_HALF, VOCAB), jnp.float32),
        pltpu.VMEM((ID_WORDS,), jnp.int32),
        pltpu.VMEM((ID_WORDS,), jnp.int32),
        pltpu.VMEM((D_HALF, CHUNK), jnp.float32),
        pltpu.VMEM((D_HALF, CHUNK), jnp.float32),
        pltpu.SemaphoreType.DMA,
        pltpu.SemaphoreType.DMA,
    ],
    compiler_params=pltpu.CompilerParams(needs_layout_passes=False),
)
def _sc_encode(img_hbm, obj_hbm, col_hbm, st_hbm, out_hbm,
               tob, tcb, tsb, idb0, idb1, ob0, ob1, sem_in, sem_out):
    _body(img_hbm, obj_hbm, col_hbm, st_hbm, out_hbm,
          tob, tcb, tsb, idb0, idb1, ob0, ob1, sem_in, sem_out)


@jax.jit
def kernel(image, object_emb, color_emb, state_emb):
    img_flat = image.astype(jnp.int32).reshape(-1)
    out = _sc_encode(img_flat, object_emb.T, color_emb.T, state_emb.T)
    return out.reshape(BATCH, EMBED_DIM, H, W)


# speedup vs baseline: 8.2135x; 3.6336x over previous
import functools

import jax
import jax.numpy as jnp
from jax import lax
from jax.experimental import pallas as pl
from jax.experimental.pallas import tpu as pltpu, tpu_sc as plsc

BATCH, H, W = 1024, 32, 32
EMBED_DIM = 64
VOCAB = 1000

NUM_CORES = 2
NUM_SUBCORES = 16
LANES = 16

D_HALF = EMBED_DIM // NUM_CORES
B_BLOCK = 128
H_HALF = H // 2
W_HALF = W // 2
D_OCT = 8
W_OCT = 8


def _body(img_hbm, obj_hbm, col_hbm, st_hbm, out_hbm,
          tob, tcb, tsb, idb0, idb1, ob0, ob1, sem_in, sem_out):
    c = lax.axis_index("c")
    s = lax.axis_index("s")
    dbase = c * D_HALF
    hh = s // 8
    b0 = (s % 8) * B_BLOCK

    idbufs = (idb0, idb1)
    obufs = (ob0, ob1)

    pltpu.sync_copy(obj_hbm.at[pl.ds(dbase, D_HALF), :], tob)
    pltpu.sync_copy(col_hbm.at[pl.ds(dbase, D_HALF), :], tcb)
    pltpu.sync_copy(st_hbm.at[pl.ds(dbase, D_HALF), :], tsb)

    def id_dmas(h, wh, slot):
        idb = idbufs[slot]
        return [
            pltpu.make_async_copy(
                img_hbm.at[h, cc, pl.ds(wh * W_HALF, W_HALF),
                           pl.ds(b0, B_BLOCK)],
                idb.at[cc], sem_in)
            for cc in range(3)
        ]

    def out_dma(h, w0, d0, slot):
        return pltpu.make_async_copy(
            obufs[slot],
            out_hbm.at[pl.ds(dbase + d0, D_OCT), h,
                       pl.ds(w0, W_OCT), pl.ds(b0, B_BLOCK)],
            sem_out)

    h_first = hh * H_HALF
    for d in id_dmas(h_first, 0, 0):
        d.start()

    inv3 = jnp.float32(1.0 / 3.0)

    def h_body(hi, _carry):
        h = hh * H_HALF + hi
        for wh in (0, 1):
            islot = wh
            for dsc in id_dmas(h, wh, islot):
                dsc.wait()
            if wh == 0:
                for dsc in id_dmas(h, 1, 1):
                    dsc.start()
            else:
                @pl.when(hi < H_HALF - 1)
                def _():
                    for dsc in id_dmas(h + 1, 0, 0):
                        dsc.start()

            idb = idbufs[islot]

            def wodo_body(wodo, _):
                wo = wodo // 2
                do2 = wodo % 2
                w0 = wh * W_HALF + wo * W_OCT
                for dop in (0, 1):
                    d0 = do2 * (2 * D_OCT) + dop * D_OCT
                    slot = dop
                    if wh == 0:
                        @pl.when((hi > 0) | (wodo > 0))
                        def _():
                            out_dma(0, 0, 0, slot).wait()
                    else:
                        out_dma(0, 0, 0, slot).wait()
                    ob = obufs[slot]

                    def bb_body(bb, _):
                        bsl = pl.ds(bb * LANES, LANES)
                        for wl in range(W_OCT):
                            wlh = wo * W_OCT + wl
                            ids_o = idb[0, wlh, bsl]
                            ids_c = idb[1, wlh, bsl]
                            ids_s = idb[2, wlh, bsl]
                            for dl in range(D_OCT):
                                dsp = d0 + jnp.full((LANES,), dl, jnp.int32)
                                vo = plsc.load_gather(tob, [dsp, ids_o])
                                vc = plsc.load_gather(tcb, [dsp, ids_c])
                                vs = plsc.load_gather(tsb, [dsp, ids_s])
                                ob[dl, wl, bsl] = (vo + vc + vs) * inv3
                        return 0

                    lax.fori_loop(0, B_BLOCK // LANES, bb_body, 0,
                                  unroll=False)
                    out_dma(h, w0, d0, slot).start()
                return 0

            lax.fori_loop(0, 4, wodo_body, 0, unroll=False)
        return 0

    lax.fori_loop(0, H_HALF, h_body, 0, unroll=False)
    out_dma(0, 0, 0, 0).wait()
    out_dma(0, 0, 0, 1).wait()


@functools.partial(
    pl.kernel,
    out_type=jax.ShapeDtypeStruct((EMBED_DIM, H, W, BATCH), jnp.float32),
    mesh=plsc.VectorSubcoreMesh(core_axis_name="c", subcore_axis_name="s",
                                num_cores=NUM_CORES,
                                num_subcores=NUM_SUBCORES),
    scratch_types=[
        pltpu.VMEM((D_HALF, VOCAB), jnp.float32),
        pltpu.VMEM((D_HALF, VOCAB), jnp.float32),
        pltpu.VMEM((D_HALF, VOCAB), jnp.float32),
        pltpu.VMEM((3, W_HALF, B_BLOCK), jnp.int32),
        pltpu.VMEM((3, W_HALF, B_BLOCK), jnp.int32),
        pltpu.VMEM((D_OCT, W_OCT, B_BLOCK), jnp.float32),
        pltpu.VMEM((D_OCT, W_OCT, B_BLOCK), jnp.float32),
        pltpu.SemaphoreType.DMA,
        pltpu.SemaphoreType.DMA,
    ],
    compiler_params=pltpu.CompilerParams(needs_layout_passes=False),
)
def _sc_encode(img_hbm, obj_hbm, col_hbm, st_hbm, out_hbm,
               tob, tcb, tsb, idb0, idb1, ob0, ob1, sem_in, sem_out):
    _body(img_hbm, obj_hbm, col_hbm, st_hbm, out_hbm,
          tob, tcb, tsb, idb0, idb1, ob0, ob1, sem_in, sem_out)


@jax.jit
def kernel(image, object_emb, color_emb, state_emb):
    img_p = jnp.transpose(image.astype(jnp.int32), (1, 3, 2, 0))
    out_p = _sc_encode(img_p, object_emb.T, color_emb.T, state_emb.T)
    return jnp.transpose(out_p, (3, 0, 1, 2))


# speedup vs baseline: 29.3863x; 3.5778x over previous
import functools

import jax
import jax.numpy as jnp
from jax import lax
from jax.experimental import pallas as pl
from jax.experimental.pallas import tpu as pltpu, tpu_sc as plsc

BATCH, H, W = 1024, 32, 32
EMBED_DIM = 64
VOCAB = 1000

NUM_CORES = 2
NUM_SUBCORES = 16
LANES = 16

D_GRP = 16
B_BLOCK = 128
W_HALF = W // 2
W_OCT = 8
ROW_PAD = 1024
NROWS = 3 * D_GRP


def _body(img_hbm, obj_hbm, col_hbm, st_hbm, out_hbm, *scratch):
    rows = scratch[:NROWS]
    idb0, idb1, ob0, ob1, sem_in, sem_out = scratch[NROWS:]
    c = lax.axis_index("c")
    s = lax.axis_index("s")
    w_id = s * NUM_CORES + c
    dbase = (w_id % 4) * D_GRP
    b0 = (w_id // 4) * B_BLOCK

    idbufs = (idb0, idb1)
    obufs = (ob0, ob1)

    descs = [
        pltpu.make_async_copy(
            src.at[pl.ds((dbase + dl) * ROW_PAD, ROW_PAD)],
            rows[cc * D_GRP + dl], sem_in)
        for cc, src in enumerate((obj_hbm, col_hbm, st_hbm))
        for dl in range(D_GRP)
    ]
    for k in range(0, NROWS, 16):
        for dsc in descs[k:k + 16]:
            dsc.start()
        for dsc in descs[k:k + 16]:
            dsc.wait()

    def id_dmas(h, wh, slot):
        idb = idbufs[slot]
        return [
            pltpu.make_async_copy(
                img_hbm.at[h, cc, pl.ds(wh * W_HALF, W_HALF),
                           pl.ds(b0, B_BLOCK)],
                idb.at[cc], sem_in)
            for cc in range(3)
        ]

    def out_dma(h, w0, slot):
        return pltpu.make_async_copy(
            obufs[slot],
            out_hbm.at[pl.ds(dbase, D_GRP), h,
                       pl.ds(w0, W_OCT), pl.ds(b0, B_BLOCK)],
            sem_out)

    for dsc in id_dmas(0, 0, 0):
        dsc.start()

    inv3 = jnp.float32(1.0 / 3.0)
    n_bb = B_BLOCK // LANES

    def h_body(hi, _carry):
        h = hi
        for wh in (0, 1):
            islot = wh
            for dsc in id_dmas(h, wh, islot):
                dsc.wait()
            if wh == 0:
                for dsc in id_dmas(h, 1, 1):
                    dsc.start()
            else:
                @pl.when(hi < H - 1)
                def _():
                    for dsc in id_dmas(h + 1, 0, 0):
                        dsc.start()

            idb = idbufs[islot]
            for wo in (0, 1):
                w0 = wh * W_HALF + wo * W_OCT
                slot = wo
                if wh == 0 and wo in (0, 1):
                    @pl.when(hi > 0)
                    def _():
                        out_dma(0, 0, slot).wait()
                else:
                    out_dma(0, 0, slot).wait()
                ob = obufs[slot]

                @plsc.parallel_loop(0, n_bb * W_OCT, 1, unroll=2)
                def bbwl_body(i):
                    bb = i // W_OCT
                    wl = i % W_OCT
                    wlh = wo * W_OCT + wl
                    bsl = pl.ds(bb * LANES, LANES)
                    ids_o = idb[0, wlh, bsl]
                    ids_c = idb[1, wlh, bsl]
                    ids_s = idb[2, wlh, bsl]
                    for dl in range(D_GRP):
                        vo = plsc.load_gather(rows[dl], [ids_o])
                        vc = plsc.load_gather(rows[D_GRP + dl], [ids_c])
                        vs = plsc.load_gather(rows[2 * D_GRP + dl], [ids_s])
                        ob[dl, wl, bsl] = (vo + vc + vs) * inv3

                out_dma(h, w0, slot).start()
        return 0

    lax.fori_loop(0, H, h_body, 0, unroll=False)
    out_dma(0, 0, 0).wait()
    out_dma(0, 0, 1).wait()


@functools.partial(
    pl.kernel,
    out_type=jax.ShapeDtypeStruct((EMBED_DIM, H, W, BATCH), jnp.float32),
    mesh=plsc.VectorSubcoreMesh(core_axis_name="c", subcore_axis_name="s",
                                num_cores=NUM_CORES,
                                num_subcores=NUM_SUBCORES),
    scratch_types=(
        [pltpu.VMEM((ROW_PAD,), jnp.float32) for _ in range(NROWS)]
        + [
            pltpu.VMEM((3, W_HALF, B_BLOCK), jnp.int32),
            pltpu.VMEM((3, W_HALF, B_BLOCK), jnp.int32),
            pltpu.VMEM((D_GRP, W_OCT, B_BLOCK), jnp.float32),
            pltpu.VMEM((D_GRP, W_OCT, B_BLOCK), jnp.float32),
            pltpu.SemaphoreType.DMA,
            pltpu.SemaphoreType.DMA,
        ]
    ),
    compiler_params=pltpu.CompilerParams(needs_layout_passes=False),
)
def _sc_encode(img_hbm, obj_hbm, col_hbm, st_hbm, out_hbm, *scratch):
    _body(img_hbm, obj_hbm, col_hbm, st_hbm, out_hbm, *scratch)


def _prep_table(t):
    return jnp.pad(t.T, ((0, 0), (0, ROW_PAD - VOCAB))).reshape(-1)


@jax.jit
def kernel(image, object_emb, color_emb, state_emb):
    img_p = jnp.transpose(image.astype(jnp.int32), (1, 3, 2, 0))
    out_p = _sc_encode(img_p, _prep_table(object_emb),
                       _prep_table(color_emb), _prep_table(state_emb))
    return jnp.transpose(out_p, (3, 0, 1, 2))
